# Initial kernel scaffold; baseline (speedup 1.0000x reference)
import jax
import jax.numpy as jnp
from jax.experimental import pallas as pl


def kernel(x_nchw, weight, gamma, beta):
    raise NotImplementedError("write your pallas kernel here")


# speedup vs baseline: 3.2989x; 3.2989x over previous
import functools

import jax
import jax.numpy as jnp
from jax import lax
from jax.experimental import pallas as pl
from jax.experimental.pallas import tpu as pltpu

_C_IN = 64
_C_OUT = 128
_EPS = 1e-5
_VMEM_LIMIT = 100 * 1024 * 1024


def _sel_moments_kernel(x_ref, sel_ref, xs_ref, s1_ref, s2_ref, *, bn):
    i = pl.program_id(0)

    @pl.when(i == 0)
    def _():
        s1_ref[...] = jnp.zeros_like(s1_ref)
        s2_ref[...] = jnp.zeros_like(s2_ref)

    sel = sel_ref[...]
    s1 = s1_ref[...]
    s2 = s2_ref[...]
    for t in range(bn):
        xc = jnp.dot(x_ref[t], sel, preferred_element_type=jnp.float32)
        xs_ref[t] = xc
        s2 = s2 + lax.dot_general(
            xc, xc, (((1,), (1,)), ((), ())),
            preferred_element_type=jnp.float32)
        s1 = s1 + jnp.sum(xc, axis=1, keepdims=True)
    s1_ref[...] = s1
    s2_ref[...] = s2


def _apply_kernel(xs_ref, ws_ref, b_ref, o_ref, *, bn):
    w = ws_ref[...]
    b = b_ref[...]
    for t in range(bn):
        o_ref[t] = (
            jnp.dot(w, xs_ref[t], preferred_element_type=jnp.float32) + b)


def kernel(x_nchw, weight, gamma, beta):
    n, cin, h, w = x_nchw.shape
    assert cin == _C_IN and h % 2 == 0 and w % 2 == 0
    ho, wo = h // 2, w // 2
    p = ho * wo
    hw = h * w
    x3 = x_nchw.astype(jnp.float32).reshape(n, cin, hw)

    pos = jnp.arange(p, dtype=jnp.int32)
    src = (2 * w) * (pos // wo) + 2 * (pos % wo)
    sel = (jnp.arange(hw, dtype=jnp.int32)[:, None] == src[None, :]).astype(
        jnp.float32)

    bn1 = 4
    xs, s1, s2 = pl.pallas_call(
        functools.partial(_sel_moments_kernel, bn=bn1),
        out_shape=(
            jax.ShapeDtypeStruct((n, cin, p), jnp.float32),
            jax.ShapeDtypeStruct((cin, 1), jnp.float32),
            jax.ShapeDtypeStruct((cin, cin), jnp.float32),
        ),
        grid=(n // bn1,),
        in_specs=[
            pl.BlockSpec((bn1, cin, hw), lambda i: (i, 0, 0)),
            pl.BlockSpec((hw, p), lambda i: (0, 0)),
        ],
        out_specs=(
            pl.BlockSpec((bn1, cin, p), lambda i: (i, 0, 0)),
            pl.BlockSpec((cin, 1), lambda i: (0, 0)),
            pl.BlockSpec((cin, cin), lambda i: (0, 0)),
        ),
        compiler_params=pltpu.CompilerParams(
            dimension_semantics=("arbitrary",),
            vmem_limit_bytes=_VMEM_LIMIT,
        ),
    )(x3, sel)

    inv_count = 1.0 / float(n * p)
    wm = weight.reshape(_C_OUT, _C_IN).astype(jnp.float32)
    mean = (wm @ s1) * inv_count
    ey2 = ((wm @ s2) * wm).sum(axis=1, keepdims=True) * inv_count
    var = jnp.maximum(ey2 - mean * mean, 0.0)
    scale = gamma.astype(jnp.float32)[:, None] * lax.rsqrt(var + _EPS)
    ws = wm * scale
    bias = beta.astype(jnp.float32)[:, None] - mean * scale

    bn2 = 4
    out = pl.pallas_call(
        functools.partial(_apply_kernel, bn=bn2),
        out_shape=jax.ShapeDtypeStruct((n, _C_OUT, p), jnp.float32),
        grid=(n // bn2,),
        in_specs=[
            pl.BlockSpec((bn2, cin, p), lambda i: (i, 0, 0)),
            pl.BlockSpec((_C_OUT, cin), lambda i: (0, 0)),
            pl.BlockSpec((_C_OUT, 1), lambda i: (0, 0)),
        ],
        out_specs=pl.BlockSpec((bn2, _C_OUT, p), lambda i: (i, 0, 0)),
        compiler_params=pltpu.CompilerParams(
            dimension_semantics=("arbitrary",),
            vmem_limit_bytes=_VMEM_LIMIT,
        ),
    )(xs, ws, bias)

    return out.reshape(n, _C_OUT, ho, wo)
